# baseline (speedup 1.0000x reference)
import functools

import jax
import jax.numpy as jnp
from jax import lax
from jax.experimental import pallas as pl
from jax.experimental.pallas import tpu as pltpu
from jax.experimental.pallas import tpu_sc as plsc

SEQ_LEN = 4096
D_MODEL = 1024
BATCH = 4

_info = plsc.get_sparse_core_info()
NUM_CORES = _info.num_cores
NUM_SUBCORES = _info.num_subcores
NUM_WORKERS = NUM_CORES * NUM_SUBCORES
LANES = _info.num_lanes

ROWS_PER_WORKER = SEQ_LEN // NUM_WORKERS
CHUNK_ROWS = 16
CHUNK_WORDS = CHUNK_ROWS * D_MODEL
CHUNKS_PER_WORKER = ROWS_PER_WORKER // CHUNK_ROWS
SEQ_WORDS = SEQ_LEN * D_MODEL


def _body(x_hbm, t_hbm, out_hbm, tbuf, ibuf):
    wid = lax.axis_index("s") * NUM_CORES + lax.axis_index("c")
    base = wid * ROWS_PER_WORKER * D_MODEL

    def chunk_body(c, _):
        toff = base + c * CHUNK_WORDS
        pltpu.sync_copy(t_hbm.at[pl.ds(toff, CHUNK_WORDS)], tbuf)

        def batch_body(b, _):
            xoff = b * SEQ_WORDS + toff
            pltpu.sync_copy(x_hbm.at[pl.ds(xoff, CHUNK_WORDS)], ibuf)

            @plsc.parallel_loop(0, CHUNK_WORDS, LANES, unroll=8)
            def add_body(i):
                sl = pl.ds(i, LANES)
                ibuf[sl] = ibuf[sl] + tbuf[sl]
            pltpu.sync_copy(ibuf, out_hbm.at[pl.ds(xoff, CHUNK_WORDS)])
            return ()

        lax.fori_loop(0, BATCH, batch_body, ())
        return ()

    lax.fori_loop(0, CHUNKS_PER_WORKER, chunk_body, ())


@jax.jit
def _pos_emb_add(x_flat, t_flat):
    mesh = plsc.VectorSubcoreMesh(core_axis_name="c", subcore_axis_name="s")
    return pl.kernel(
        _body,
        out_type=jax.ShapeDtypeStruct((BATCH * SEQ_WORDS,), jnp.float32),
        mesh=mesh,
        scratch_types=[
            pltpu.VMEM((CHUNK_WORDS,), jnp.float32),
            pltpu.VMEM((CHUNK_WORDS,), jnp.float32),
        ],
    )(x_flat, t_flat)


def kernel(inputs, pos_table):
    out = _pos_emb_add(inputs.reshape(-1), pos_table.reshape(-1))
    return out.reshape(BATCH, SEQ_LEN, D_MODEL)


# speedup vs baseline: 1.2798x; 1.2798x over previous
import jax
import jax.numpy as jnp
from jax import lax
from jax.experimental import pallas as pl
from jax.experimental.pallas import tpu as pltpu
from jax.experimental.pallas import tpu_sc as plsc

SEQ_LEN = 4096
D_MODEL = 1024
BATCH = 4

_info = plsc.get_sparse_core_info()
NUM_CORES = _info.num_cores
NUM_SUBCORES = _info.num_subcores
NUM_WORKERS = NUM_CORES * NUM_SUBCORES
LANES = _info.num_lanes

ROWS_PER_WORKER = SEQ_LEN // NUM_WORKERS
CHUNK_ROWS = 16
CHUNK_WORDS = CHUNK_ROWS * D_MODEL
NUM_CHUNKS = ROWS_PER_WORKER // CHUNK_ROWS
SEQ_WORDS = SEQ_LEN * D_MODEL


def _body(x_hbm, t_hbm, out_hbm,
          ib0, ib1, ob0, ob1, tb0, tb1,
          in_s0, in_s1, out_s0, out_s1, t_s0, t_s1):
    wid = lax.axis_index("s") * NUM_CORES + lax.axis_index("c")
    base = wid * ROWS_PER_WORKER * D_MODEL

    ibufs = (ib0, ib1)
    obufs = (ob0, ob1)
    tbufs = (tb0, tb1)
    in_sems = (in_s0, in_s1)
    out_sems = (out_s0, out_s1)
    t_sems = (t_s0, t_s1)

    def t_slice(chunk):
        return t_hbm.at[pl.ds(base + chunk * CHUNK_WORDS, CHUNK_WORDS)]

    def x_slice(chunk, b):
        return x_hbm.at[pl.ds(b * SEQ_WORDS + base + chunk * CHUNK_WORDS,
                              CHUNK_WORDS)]

    def o_slice(chunk, b):
        return out_hbm.at[pl.ds(b * SEQ_WORDS + base + chunk * CHUNK_WORDS,
                                CHUNK_WORDS)]

    pltpu.make_async_copy(x_slice(0, 0), ib0, in_s0).start()
    pltpu.make_async_copy(x_slice(0, 1), ib1, in_s1).start()
    pltpu.make_async_copy(t_slice(0), tb0, t_s0).start()
    pltpu.make_async_copy(t_slice(1), tb1, t_s1).start()

    def chunk_pair(it, _):
        for cp in (0, 1):
            chunk = 2 * it + cp
            pltpu.make_async_copy(t_slice(chunk), tbufs[cp], t_sems[cp]).wait()

            for b in range(BATCH):
                p = b % 2
                pltpu.make_async_copy(x_slice(chunk, b), ibufs[p],
                                      in_sems[p]).wait()
                ob_prev = o_slice(chunk - (1 if b < 2 else 0), (b + 2) % 4)

                def wait_out():
                    pltpu.make_async_copy(obufs[p], ob_prev,
                                          out_sems[p]).wait()

                if cp == 0 and b < 2:
                    pl.when(it > 0)(wait_out)
                else:
                    wait_out()

                ib, ob, tb = ibufs[p], obufs[p], tbufs[cp]

                @plsc.parallel_loop(0, CHUNK_WORDS, LANES, unroll=8)
                def add_body(i):
                    sl = pl.ds(i, LANES)
                    ob[sl] = ib[sl] + tb[sl]

                pltpu.make_async_copy(obufs[p], o_slice(chunk, b),
                                      out_sems[p]).start()

                nchunk = chunk + (1 if b >= 2 else 0)
                nb = (b + 2) % 4

                def start_in():
                    pltpu.make_async_copy(x_slice(nchunk, nb), ibufs[p],
                                          in_sems[p]).start()

                if b >= 2:
                    pl.when(chunk < NUM_CHUNKS - 1)(start_in)
                else:
                    start_in()

            def start_t():
                pltpu.make_async_copy(t_slice(chunk + 2), tbufs[cp],
                                      t_sems[cp]).start()

            pl.when(chunk < NUM_CHUNKS - 2)(start_t)
        return ()

    lax.fori_loop(0, NUM_CHUNKS // 2, chunk_pair, ())

    pltpu.make_async_copy(ob0, o_slice(NUM_CHUNKS - 1, 2), out_s0).wait()
    pltpu.make_async_copy(ob1, o_slice(NUM_CHUNKS - 1, 3), out_s1).wait()


@jax.jit
def _pos_emb_add(x_flat, t_flat):
    mesh = plsc.VectorSubcoreMesh(core_axis_name="c", subcore_axis_name="s")
    return pl.kernel(
        _body,
        out_type=jax.ShapeDtypeStruct((BATCH * SEQ_WORDS,), jnp.float32),
        mesh=mesh,
        scratch_types=[
            pltpu.VMEM((CHUNK_WORDS,), jnp.float32),
            pltpu.VMEM((CHUNK_WORDS,), jnp.float32),
            pltpu.VMEM((CHUNK_WORDS,), jnp.float32),
            pltpu.VMEM((CHUNK_WORDS,), jnp.float32),
            pltpu.VMEM((CHUNK_WORDS,), jnp.float32),
            pltpu.VMEM((CHUNK_WORDS,), jnp.float32),
            pltpu.SemaphoreType.DMA,
            pltpu.SemaphoreType.DMA,
            pltpu.SemaphoreType.DMA,
            pltpu.SemaphoreType.DMA,
            pltpu.SemaphoreType.DMA,
            pltpu.SemaphoreType.DMA,
        ],
    )(x_flat, t_flat)


def kernel(inputs, pos_table):
    out = _pos_emb_add(inputs.reshape(-1), pos_table.reshape(-1))
    return out.reshape(BATCH, SEQ_LEN, D_MODEL)


# speedup vs baseline: 3.5135x; 2.7454x over previous
import jax
import jax.numpy as jnp
from jax import lax
from jax.experimental import pallas as pl
from jax.experimental.pallas import tpu as pltpu
from jax.experimental.pallas import tpu_sc as plsc

SEQ_LEN = 4096
D_MODEL = 1024
BATCH = 4

_info = plsc.get_sparse_core_info()
NUM_CORES = _info.num_cores
NUM_SUBCORES = _info.num_subcores
NUM_WORKERS = NUM_CORES * NUM_SUBCORES
LANES = _info.num_lanes

ROWS_PER_WORKER = SEQ_LEN // NUM_WORKERS
CHUNK_ROWS = 16
CHUNK_WORDS = CHUNK_ROWS * D_MODEL
NUM_CHUNKS = ROWS_PER_WORKER // CHUNK_ROWS


def _body(x_hbm, t_hbm, out_hbm,
          ib0, ib1, ob0, ob1, tb0, tb1,
          in_s0, in_s1, out_s0, out_s1, t_s0, t_s1):
    wid = lax.axis_index("s") * NUM_CORES + lax.axis_index("c")
    base_row = wid * ROWS_PER_WORKER

    ibufs = (ib0, ib1)
    obufs = (ob0, ob1)
    tbufs = (tb0, tb1)
    in_sems = (in_s0, in_s1)
    out_sems = (out_s0, out_s1)
    t_sems = (t_s0, t_s1)

    def t_slice(chunk):
        return t_hbm.at[pl.ds(base_row + chunk * CHUNK_ROWS, CHUNK_ROWS), :]

    def x_slice(chunk, b):
        return x_hbm.at[b, pl.ds(base_row + chunk * CHUNK_ROWS, CHUNK_ROWS), :]

    def o_slice(chunk, b):
        return out_hbm.at[b, pl.ds(base_row + chunk * CHUNK_ROWS, CHUNK_ROWS), :]

    pltpu.make_async_copy(x_slice(0, 0), ib0, in_s0).start()
    pltpu.make_async_copy(x_slice(0, 1), ib1, in_s1).start()
    pltpu.make_async_copy(t_slice(0), tb0, t_s0).start()
    pltpu.make_async_copy(t_slice(1), tb1, t_s1).start()

    def chunk_pair(it, _):
        for cp in (0, 1):
            chunk = 2 * it + cp
            pltpu.make_async_copy(t_slice(chunk), tbufs[cp], t_sems[cp]).wait()

            for b in range(BATCH):
                p = b % 2
                pltpu.make_async_copy(x_slice(chunk, b), ibufs[p],
                                      in_sems[p]).wait()
                ob_prev = o_slice(chunk - (1 if b < 2 else 0), (b + 2) % 4)

                def wait_out():
                    pltpu.make_async_copy(obufs[p], ob_prev,
                                          out_sems[p]).wait()

                if cp == 0 and b < 2:
                    pl.when(it > 0)(wait_out)
                else:
                    wait_out()

                ib, ob, tb = ibufs[p], obufs[p], tbufs[cp]

                @plsc.parallel_loop(0, CHUNK_WORDS, LANES, unroll=8)
                def add_body(i):
                    r = i // D_MODEL
                    c = i % D_MODEL
                    sl = pl.ds(c, LANES)
                    ob[r, sl] = ib[r, sl] + tb[r, sl]

                pltpu.make_async_copy(obufs[p], o_slice(chunk, b),
                                      out_sems[p]).start()

                nchunk = chunk + (1 if b >= 2 else 0)
                nb = (b + 2) % 4

                def start_in():
                    pltpu.make_async_copy(x_slice(nchunk, nb), ibufs[p],
                                          in_sems[p]).start()

                if b >= 2:
                    pl.when(chunk < NUM_CHUNKS - 1)(start_in)
                else:
                    start_in()

            def start_t():
                pltpu.make_async_copy(t_slice(chunk + 2), tbufs[cp],
                                      t_sems[cp]).start()

            pl.when(chunk < NUM_CHUNKS - 2)(start_t)
        return ()

    lax.fori_loop(0, NUM_CHUNKS // 2, chunk_pair, ())

    pltpu.make_async_copy(ob0, o_slice(NUM_CHUNKS - 1, 2), out_s0).wait()
    pltpu.make_async_copy(ob1, o_slice(NUM_CHUNKS - 1, 3), out_s1).wait()


@jax.jit
def _pos_emb_add(x, t):
    mesh = plsc.VectorSubcoreMesh(core_axis_name="c", subcore_axis_name="s")
    buf = pltpu.VMEM((CHUNK_ROWS, D_MODEL), jnp.float32)
    return pl.kernel(
        _body,
        out_type=jax.ShapeDtypeStruct((BATCH, SEQ_LEN, D_MODEL), jnp.float32),
        mesh=mesh,
        scratch_types=[
            buf, buf, buf, buf, buf, buf,
            pltpu.SemaphoreType.DMA,
            pltpu.SemaphoreType.DMA,
            pltpu.SemaphoreType.DMA,
            pltpu.SemaphoreType.DMA,
            pltpu.SemaphoreType.DMA,
            pltpu.SemaphoreType.DMA,
        ],
    )(x, t)


def kernel(inputs, pos_table):
    return _pos_emb_add(inputs, pos_table)
